# baseline (speedup 1.0000x reference)
import functools

import jax
import jax.numpy as jnp
from jax import lax
from jax.experimental import pallas as pl
from jax.experimental.pallas import tpu as pltpu
from jax.experimental.pallas import tpu_sc as plsc

_BATCH = 16384
_K = 32
_NC = 2
_NS = 16
_NW = _NC * _NS
_CHUNK = _BATCH // _NW
_IDX_BLKS = _CHUNK // 128
_LANES = 16
_NBLK = _CHUNK // _LANES


def _mf_body(x_hbm, w_hbm, h_hbm, out_hbm, xv, uidx, iidx, uv, vv, outv, sem):
    wid = lax.axis_index("s") * _NC + lax.axis_index("c")
    base = wid * _CHUNK

    pltpu.sync_copy(x_hbm.at[pl.ds(base, _CHUNK)], xv)

    lanes = lax.iota(jnp.int32, _LANES)
    zeros = jnp.zeros((_LANES,), jnp.int32)
    ones = jnp.ones((_LANES,), jnp.int32)

    def deinterleave(b, _):
        rows = lanes + b * _LANES
        j = b // 8
        o = (b % 8) * _LANES
        uidx[j, pl.ds(o, _LANES)] = plsc.load_gather(xv, [rows, zeros])
        iidx[j, pl.ds(o, _LANES)] = plsc.load_gather(xv, [rows, ones])
        return 0

    lax.fori_loop(0, _NBLK, deinterleave, 0)

    handles = []
    for j in range(_IDX_BLKS):
        handles.append(
            pltpu.async_copy(w_hbm.at[uidx.at[j]], uv.at[pl.ds(j * 128, 128)], sem))
    for j in range(_IDX_BLKS):
        handles.append(
            pltpu.async_copy(h_hbm.at[iidx.at[j]], vv.at[pl.ds(j * 128, 128)], sem))
    for h in handles:
        h.wait()

    def compute(b, _):
        rows = lanes + b * _LANES
        acc = jnp.zeros((_LANES,), jnp.float32)
        for k in range(_K):
            ck = jnp.full((_LANES,), k, jnp.int32)
            acc += plsc.load_gather(uv, [rows, ck]) * plsc.load_gather(vv, [rows, ck])
        outv[pl.ds(b * _LANES, _LANES)] = 1.0 / (1.0 + jnp.exp(-acc))
        return 0

    lax.fori_loop(0, _NBLK, compute, 0)

    pltpu.sync_copy(outv, out_hbm.at[pl.ds(base, _CHUNK)])


_mf = functools.partial(
    pl.kernel,
    out_type=jax.ShapeDtypeStruct((_BATCH,), jnp.float32),
    mesh=plsc.VectorSubcoreMesh(core_axis_name="c", subcore_axis_name="s"),
    scratch_types=[
        pltpu.VMEM((_CHUNK, 2), jnp.int32),
        pltpu.VMEM((_IDX_BLKS, 128), jnp.int32),
        pltpu.VMEM((_IDX_BLKS, 128), jnp.int32),
        pltpu.VMEM((_CHUNK, _K), jnp.float32),
        pltpu.VMEM((_CHUNK, _K), jnp.float32),
        pltpu.VMEM((_CHUNK,), jnp.float32),
        pltpu.SemaphoreType.DMA,
    ],
    compiler_params=pltpu.CompilerParams(
        needs_layout_passes=False, use_tc_tiling_on_sc=False),
)(_mf_body)


@jax.jit
def kernel(x, W, H):
    return _mf(x.astype(jnp.int32), W, H)
